# baseline (speedup 1.0000x reference)
import functools

import jax
import jax.numpy as jnp
from jax import lax
from jax.experimental import pallas as pl
from jax.experimental.pallas import tpu as pltpu
from jax.experimental.pallas import tpu_sc as plsc

N = 10000
E = 320000
F = 128
POS = 3
MSGW = 32
H1 = 528
SCH = 128
NCHT = E // SCH
NW = 32
NEG = 16
NCG = 2
CCG = 10
NPAD = 10240
BE = 1280
BN = 2048


def _silu(v):
    return v * jax.nn.sigmoid(v)


def _sc_gather(feats, coors_flat, dst, src):
    info = plsc.get_sparse_core_info()
    nc = info.num_cores
    mesh = plsc.VectorSubcoreMesh(core_axis_name="c", subcore_axis_name="s")
    npairs = (NCHT // NW) // 2
    ntail = NCHT - NW * 2 * npairs
    ngr = SCH // 16

    @functools.partial(
        pl.kernel,
        mesh=mesh,
        compiler_params=pltpu.CompilerParams(needs_layout_passes=False),
        out_type=[jax.ShapeDtypeStruct((E, F), jnp.float32),
                  jax.ShapeDtypeStruct((E, F), jnp.float32),
                  jax.ShapeDtypeStruct((4, E), jnp.float32)],
        scratch_types=[pltpu.VMEM((4 * NPAD,), jnp.float32)]
        + [pltpu.VMEM((SCH,), jnp.int32) for _ in range(4)]
        + [pltpu.VMEM((SCH, F), jnp.float32) for _ in range(4)]
        + [pltpu.VMEM((4, SCH), jnp.float32) for _ in range(2)]
        + [pltpu.SemaphoreType.DMA for _ in range(4)],
    )
    def k(feats_hbm, coor_hbm, dst_hbm, src_hbm, gd_hbm, gs_hbm, rr_hbm,
          coor_v, ixd_a, ixs_a, ixd_b, ixs_b, rod_a, ros_a, rod_b, ros_b,
          rr_a, rr_b, sem_a, sem_b, sem_wa, sem_wb):
        wid = lax.axis_index("s") * nc + lax.axis_index("c")
        pltpu.sync_copy(coor_hbm, coor_v)

        def relrd(ixd, ixs, rr_v):
            for g in range(ngr):
                dvec = ixd[pl.ds(g * 16, 16)]
                svec = ixs[pl.ds(g * 16, 16)]
                rd = jnp.zeros((16,), jnp.float32)
                for d in range(POS):
                    cd = plsc.load_gather(coor_v, [dvec + d * NPAD])
                    cs = plsc.load_gather(coor_v, [svec + d * NPAD])
                    rel = cs - cd
                    rr_v[d, pl.ds(g * 16, 16)] = rel
                    rd = rd + rel * rel
                rr_v[POS, pl.ds(g * 16, 16)] = rd

        def fetch(chunk, ixd, ixs, rod, ros, sem):
            off = chunk * SCH
            pltpu.sync_copy(dst_hbm.at[pl.ds(off, SCH)], ixd)
            pltpu.sync_copy(src_hbm.at[pl.ds(off, SCH)], ixs)
            cp_d = pltpu.async_copy(feats_hbm.at[ixd], rod, sem)
            cp_s = pltpu.async_copy(feats_hbm.at[ixs], ros, sem)
            return cp_d, cp_s

        def flush(chunk, rod, ros, rr_v, sem_w):
            off = chunk * SCH
            wd = pltpu.async_copy(rod, gd_hbm.at[pl.ds(off, SCH)], sem_w)
            ws = pltpu.async_copy(ros, gs_hbm.at[pl.ds(off, SCH)], sem_w)
            pltpu.sync_copy(rr_v, rr_hbm.at[:, pl.ds(off, SCH)])
            return wd, ws

        def body(i, carry):
            ca = wid + (2 * i) * NW
            cb = wid + (2 * i + 1) * NW
            ga_d, ga_s = fetch(ca, ixd_a, ixs_a, rod_a, ros_a, sem_a)
            gb_d, gb_s = fetch(cb, ixd_b, ixs_b, rod_b, ros_b, sem_b)
            relrd(ixd_a, ixs_a, rr_a)
            ga_d.wait()
            ga_s.wait()
            wa_d, wa_s = flush(ca, rod_a, ros_a, rr_a, sem_wa)
            relrd(ixd_b, ixs_b, rr_b)
            gb_d.wait()
            gb_s.wait()
            wb_d, wb_s = flush(cb, rod_b, ros_b, rr_b, sem_wb)
            wa_d.wait()
            wa_s.wait()
            wb_d.wait()
            wb_s.wait()
            return carry

        lax.fori_loop(0, npairs, body, 0)

        @pl.when(wid < ntail)
        def _tail():
            ct = NW * 2 * npairs + wid
            ga_d, ga_s = fetch(ct, ixd_a, ixs_a, rod_a, ros_a, sem_a)
            relrd(ixd_a, ixs_a, rr_a)
            ga_d.wait()
            ga_s.wait()
            wa_d, wa_s = flush(ct, rod_a, ros_a, rr_a, sem_wa)
            wa_d.wait()
            wa_s.wait()

    return k(feats, coors_flat, dst, src)


def _sc_scatter(msg_t, dst, zeros_acc):
    info = plsc.get_sparse_core_info()
    nc = info.num_cores
    mesh = plsc.VectorSubcoreMesh(core_axis_name="c", subcore_axis_name="s")
    ngr = SCH // 16
    npairs = (NCHT // NEG) // 2
    ntail = NCHT - NEG * 2 * npairs

    @functools.partial(
        pl.kernel,
        mesh=mesh,
        compiler_params=pltpu.CompilerParams(needs_layout_passes=False),
        out_type=jax.ShapeDtypeStruct((NW, CCG * NPAD), jnp.float32),
        scratch_types=[pltpu.VMEM((SCH,), jnp.int32),
                       pltpu.VMEM((SCH,), jnp.int32),
                       pltpu.VMEM((MSGW, SCH), jnp.float32),
                       pltpu.VMEM((MSGW, SCH), jnp.float32),
                       pltpu.VMEM((CCG * NPAD,), jnp.float32),
                       pltpu.SemaphoreType.DMA,
                       pltpu.SemaphoreType.DMA],
    )
    def k(msg_hbm, dst_hbm, z_hbm, out_hbm, ix_a, ix_b, col_a, col_b, acc,
          sem_a, sem_b):
        wid = lax.axis_index("s") * nc + lax.axis_index("c")
        eg = wid // NCG
        cg = wid % NCG
        cbase = cg * CCG
        pltpu.sync_copy(z_hbm, acc)

        def fetch(chunk, ix, col, sem):
            off = chunk * SCH
            ci = pltpu.async_copy(dst_hbm.at[pl.ds(off, SCH)], ix, sem)
            cm = pltpu.async_copy(msg_hbm.at[:, pl.ds(off, SCH)], col, sem)
            return ci, cm

        def scatter(ix, col):
            for g in range(ngr):
                dvec = ix[pl.ds(g * 16, 16)]
                for c in range(CCG):
                    vals = col[cbase + c, pl.ds(g * 16, 16)]
                    plsc.addupdate_scatter(acc, [dvec + c * NPAD], vals)

        fetch(eg, ix_a, col_a, sem_a)

        def wait_fetch(ix, col, sem):
            pltpu.make_async_copy(dst_hbm.at[pl.ds(0, SCH)], ix, sem).wait()
            pltpu.make_async_copy(msg_hbm.at[:, pl.ds(0, SCH)], col,
                                  sem).wait()

        def body(i, carry):
            cb = eg + (2 * i + 1) * NEG
            cn = jnp.minimum(eg + (2 * i + 2) * NEG, NCHT - 1)
            wait_fetch(ix_a, col_a, sem_a)
            fetch(cb, ix_b, col_b, sem_b)
            scatter(ix_a, col_a)
            wait_fetch(ix_b, col_b, sem_b)
            fetch(cn, ix_a, col_a, sem_a)
            scatter(ix_b, col_b)
            return carry

        lax.fori_loop(0, npairs, body, 0)
        wait_fetch(ix_a, col_a, sem_a)

        @pl.when(eg < ntail)
        def _tail():
            scatter(ix_a, col_a)

        pltpu.sync_copy(acc, out_hbm.at[wid])

    return k(msg_t, dst, zeros_acc)


def _tc_edge(gd, gs, rr, eap, wd, ws, wea, wdr, b1, w2, b2, wc1, bc1,
             wc2, bc2):
    nb = E // BE

    def body(gd_ref, gs_ref, rr_ref, ea_ref, wd_ref, ws_ref, wea_ref,
             wdr_ref, b1_ref, w2_ref, b2_ref, wc1_ref, bc1_ref, wc2_ref,
             bc2_ref, out_ref):
        rrt = rr_ref[...].T
        rel = rrt[:, :POS]
        rd = rrt[:, POS:POS + 1]
        h = (jnp.dot(gd_ref[...], wd_ref[...],
                     preferred_element_type=jnp.float32)
             + jnp.dot(gs_ref[...], ws_ref[...],
                       preferred_element_type=jnp.float32)
             + jnp.dot(ea_ref[...], wea_ref[...],
                       preferred_element_type=jnp.float32)
             + rd * wdr_ref[...]
             + b1_ref[...])
        h = _silu(h)
        m = _silu(jnp.dot(h, w2_ref[...], preferred_element_type=jnp.float32)
                  + b2_ref[...])
        cw = _silu(jnp.dot(m, wc1_ref[...], preferred_element_type=jnp.float32)
                   + bc1_ref[...])
        cw = jnp.dot(cw, wc2_ref[...], preferred_element_type=jnp.float32) \
            + bc2_ref[...]
        out_ref[...] = jnp.concatenate(
            [m, cw * rel, jnp.zeros((BE, MSGW - 19), jnp.float32)],
            axis=1).T

    full = lambda shape: pl.BlockSpec(shape, lambda i: (0,) * len(shape))
    return pl.pallas_call(
        body,
        grid=(nb,),
        in_specs=[
            pl.BlockSpec((BE, F), lambda i: (i, 0)),
            pl.BlockSpec((BE, F), lambda i: (i, 0)),
            pl.BlockSpec((4, BE), lambda i: (0, i)),
            pl.BlockSpec((BE, 8), lambda i: (i, 0)),
            full((F, H1)), full((F, H1)), full((8, H1)), full((1, H1)),
            full((1, H1)), full((H1, 16)), full((1, 16)),
            full((16, 64)), full((1, 64)), full((64, 1)), full((1, 1)),
        ],
        out_specs=pl.BlockSpec((MSGW, BE), lambda i: (0, i)),
        out_shape=jax.ShapeDtypeStruct((MSGW, E), jnp.float32),
    )(gd, gs, rr, eap, wd, ws, wea, wdr, b1, w2, b2, wc1, bc1, wc2, bc2)


def _tc_node(feats, coors2d, acc3, wn1, bn1, wn2, bn2):
    nb = -(-N // BN)

    def body(f_ref, c_ref, acc_ref, wn1_ref, bn1_ref, wn2_ref, bn2_ref,
             fo_ref, co_ref):
        a0 = acc_ref[0]
        a1 = acc_ref[1]
        for g in range(1, NEG):
            a0 = a0 + acc_ref[g * NCG]
            a1 = a1 + acc_ref[g * NCG + 1]
        a = jnp.concatenate([a0.T, a1.T], axis=1)
        feats = f_ref[...]
        nin = jnp.concatenate([feats, a[:, :16]], axis=1)
        hid = _silu(jnp.dot(nin, wn1_ref[...],
                            preferred_element_type=jnp.float32) + bn1_ref[...])
        hid = jnp.dot(hid, wn2_ref[...],
                      preferred_element_type=jnp.float32) + bn2_ref[...]
        fo_ref[...] = feats + hid
        co_ref[...] = c_ref[...] + jnp.concatenate(
            [a[:, 16:19], jnp.zeros((BN, 1), jnp.float32)], axis=1).T

    full = lambda shape: pl.BlockSpec(shape, lambda i: (0,) * len(shape))
    return pl.pallas_call(
        body,
        grid=(nb,),
        in_specs=[
            pl.BlockSpec((BN, F), lambda i: (i, 0)),
            pl.BlockSpec((4, BN), lambda i: (0, i)),
            pl.BlockSpec((NW, CCG, BN), lambda i: (0, 0, i)),
            full((F + 16, 2 * F)), full((1, 2 * F)),
            full((2 * F, F)), full((1, F)),
        ],
        out_specs=[pl.BlockSpec((BN, F), lambda i: (i, 0)),
                   pl.BlockSpec((4, BN), lambda i: (0, i))],
        out_shape=[jax.ShapeDtypeStruct((N, F), jnp.float32),
                   jax.ShapeDtypeStruct((4, NPAD), jnp.float32)],
    )(feats, coors2d, acc3, wn1, bn1, wn2, bn2)


def _pad_weights(p):
    w1 = jnp.pad(p["We1"], ((0, 0), (0, H1 - p["We1"].shape[1])))
    wd = w1[:F]
    ws = w1[F:2 * F]
    wea = jnp.pad(w1[2 * F:2 * F + 4], ((0, 4), (0, 0)))
    wdr = w1[2 * F + 4:2 * F + 5]
    b1 = jnp.pad(p["be1"], (0, H1 - p["be1"].shape[0])).reshape(1, H1)
    w2 = jnp.pad(p["We2"], ((0, H1 - p["We2"].shape[0]), (0, 0)))
    return dict(wd=wd, ws=ws, wea=wea, wdr=wdr, b1=b1, w2=w2,
                b2=p["be2"].reshape(1, -1),
                wc1=p["Wc1"], bc1=p["bc1"].reshape(1, -1),
                wc2=p["Wc2"], bc2=p["bc2"].reshape(1, -1),
                wn1=p["Wn1"], bn1=p["bn1"].reshape(1, -1),
                wn2=p["Wn2"], bn2=p["bn2"].reshape(1, -1))


def kernel(x, edge_index, batch, edge_attr, params):
    src = edge_index[0]
    dst = edge_index[1]
    feats = x[:, POS:]
    coors2d = jnp.pad(
        jnp.concatenate([x[:, :POS].T, jnp.zeros((1, N), jnp.float32)],
                        axis=0), ((0, 0), (0, NPAD - N)))
    eap = jnp.pad(edge_attr, ((0, 0), (0, 4)))
    zeros_acc = jnp.zeros((CCG * NPAD,), jnp.float32)
    for p in params:
        w = _pad_weights(p)
        gd, gs, rr = _sc_gather(feats, coors2d.reshape(4 * NPAD), dst, src)
        msg_t = _tc_edge(gd, gs, rr, eap, w["wd"], w["ws"], w["wea"],
                         w["wdr"], w["b1"], w["w2"], w["b2"], w["wc1"],
                         w["bc1"], w["wc2"], w["bc2"])
        acc = _sc_scatter(msg_t, dst, zeros_acc)
        acc3 = acc.reshape(NW, CCG, NPAD)
        feats, coors2d = _tc_node(feats, coors2d, acc3, w["wn1"], w["bn1"],
                                  w["wn2"], w["bn2"])
    return jnp.concatenate([coors2d[:POS, :N].T, feats], axis=1)


# speedup vs baseline: 1.0104x; 1.0104x over previous
import functools

import jax
import jax.numpy as jnp
from jax import lax
from jax.experimental import pallas as pl
from jax.experimental.pallas import tpu as pltpu
from jax.experimental.pallas import tpu_sc as plsc

N = 10000
E = 320000
F = 128
POS = 3
MSGW = 32
H1 = 528
SCH = 128
NCHT = E // SCH
NW = 32
NEG = 16
NCG = 2
CCG = 10
NPAD = 10240
BE = 2560
BN = 2048


def _silu(v):
    return v * jax.nn.sigmoid(v)


def _sc_gather(feats, coors_flat, dst, src):
    info = plsc.get_sparse_core_info()
    nc = info.num_cores
    mesh = plsc.VectorSubcoreMesh(core_axis_name="c", subcore_axis_name="s")
    npairs = (NCHT // NW) // 2
    ntail = NCHT - NW * 2 * npairs
    ngr = SCH // 16

    @functools.partial(
        pl.kernel,
        mesh=mesh,
        compiler_params=pltpu.CompilerParams(needs_layout_passes=False),
        out_type=[jax.ShapeDtypeStruct((E, F), jnp.float32),
                  jax.ShapeDtypeStruct((E, F), jnp.float32),
                  jax.ShapeDtypeStruct((4, E), jnp.float32)],
        scratch_types=[pltpu.VMEM((4 * NPAD,), jnp.float32)]
        + [pltpu.VMEM((SCH,), jnp.int32) for _ in range(4)]
        + [pltpu.VMEM((SCH, F), jnp.float32) for _ in range(4)]
        + [pltpu.VMEM((4, SCH), jnp.float32) for _ in range(2)]
        + [pltpu.SemaphoreType.DMA for _ in range(4)],
    )
    def k(feats_hbm, coor_hbm, dst_hbm, src_hbm, gd_hbm, gs_hbm, rr_hbm,
          coor_v, ixd_a, ixs_a, ixd_b, ixs_b, rod_a, ros_a, rod_b, ros_b,
          rr_a, rr_b, sem_a, sem_b, sem_wa, sem_wb):
        wid = lax.axis_index("s") * nc + lax.axis_index("c")
        pltpu.sync_copy(coor_hbm, coor_v)

        def relrd(ixd, ixs, rr_v):
            for g in range(ngr):
                dvec = ixd[pl.ds(g * 16, 16)]
                svec = ixs[pl.ds(g * 16, 16)]
                rd = jnp.zeros((16,), jnp.float32)
                for d in range(POS):
                    cd = plsc.load_gather(coor_v, [dvec + d * NPAD])
                    cs = plsc.load_gather(coor_v, [svec + d * NPAD])
                    rel = cs - cd
                    rr_v[d, pl.ds(g * 16, 16)] = rel
                    rd = rd + rel * rel
                rr_v[POS, pl.ds(g * 16, 16)] = rd

        def fetch(chunk, ixd, ixs, rod, ros, sem):
            off = chunk * SCH
            pltpu.sync_copy(dst_hbm.at[pl.ds(off, SCH)], ixd)
            pltpu.sync_copy(src_hbm.at[pl.ds(off, SCH)], ixs)
            cp_d = pltpu.async_copy(feats_hbm.at[ixd], rod, sem)
            cp_s = pltpu.async_copy(feats_hbm.at[ixs], ros, sem)
            return cp_d, cp_s

        def flush(chunk, rod, ros, rr_v, sem_w):
            off = chunk * SCH
            wd = pltpu.async_copy(rod, gd_hbm.at[pl.ds(off, SCH)], sem_w)
            ws = pltpu.async_copy(ros, gs_hbm.at[pl.ds(off, SCH)], sem_w)
            pltpu.sync_copy(rr_v, rr_hbm.at[:, pl.ds(off, SCH)])
            return wd, ws

        def body(i, carry):
            ca = wid + (2 * i) * NW
            cb = wid + (2 * i + 1) * NW
            ga_d, ga_s = fetch(ca, ixd_a, ixs_a, rod_a, ros_a, sem_a)
            gb_d, gb_s = fetch(cb, ixd_b, ixs_b, rod_b, ros_b, sem_b)
            relrd(ixd_a, ixs_a, rr_a)
            ga_d.wait()
            ga_s.wait()
            wa_d, wa_s = flush(ca, rod_a, ros_a, rr_a, sem_wa)
            relrd(ixd_b, ixs_b, rr_b)
            gb_d.wait()
            gb_s.wait()
            wb_d, wb_s = flush(cb, rod_b, ros_b, rr_b, sem_wb)
            wa_d.wait()
            wa_s.wait()
            wb_d.wait()
            wb_s.wait()
            return carry

        lax.fori_loop(0, npairs, body, 0)

        @pl.when(wid < ntail)
        def _tail():
            ct = NW * 2 * npairs + wid
            ga_d, ga_s = fetch(ct, ixd_a, ixs_a, rod_a, ros_a, sem_a)
            relrd(ixd_a, ixs_a, rr_a)
            ga_d.wait()
            ga_s.wait()
            wa_d, wa_s = flush(ct, rod_a, ros_a, rr_a, sem_wa)
            wa_d.wait()
            wa_s.wait()

    return k(feats, coors_flat, dst, src)


def _sc_scatter(msg_t, dst, zeros_acc):
    info = plsc.get_sparse_core_info()
    nc = info.num_cores
    mesh = plsc.VectorSubcoreMesh(core_axis_name="c", subcore_axis_name="s")
    ngr = SCH // 16
    npairs = (NCHT // NEG) // 2
    ntail = NCHT - NEG * 2 * npairs

    @functools.partial(
        pl.kernel,
        mesh=mesh,
        compiler_params=pltpu.CompilerParams(needs_layout_passes=False),
        out_type=jax.ShapeDtypeStruct((NW, CCG * NPAD), jnp.float32),
        scratch_types=[pltpu.VMEM((SCH,), jnp.int32),
                       pltpu.VMEM((SCH,), jnp.int32),
                       pltpu.VMEM((MSGW, SCH), jnp.float32),
                       pltpu.VMEM((MSGW, SCH), jnp.float32),
                       pltpu.VMEM((CCG * NPAD,), jnp.float32),
                       pltpu.SemaphoreType.DMA,
                       pltpu.SemaphoreType.DMA],
    )
    def k(msg_hbm, dst_hbm, z_hbm, out_hbm, ix_a, ix_b, col_a, col_b, acc,
          sem_a, sem_b):
        wid = lax.axis_index("s") * nc + lax.axis_index("c")
        eg = wid // NCG
        cg = wid % NCG
        cbase = cg * CCG
        pltpu.sync_copy(z_hbm, acc)

        def fetch(chunk, ix, col, sem):
            off = chunk * SCH
            ci = pltpu.async_copy(dst_hbm.at[pl.ds(off, SCH)], ix, sem)
            cm = pltpu.async_copy(msg_hbm.at[:, pl.ds(off, SCH)], col, sem)
            return ci, cm

        def scatter(ix, col):
            for g in range(ngr):
                dvec = ix[pl.ds(g * 16, 16)]
                for c in range(CCG):
                    vals = col[cbase + c, pl.ds(g * 16, 16)]
                    plsc.addupdate_scatter(acc, [dvec + c * NPAD], vals)

        fetch(eg, ix_a, col_a, sem_a)

        def wait_fetch(ix, col, sem):
            pltpu.make_async_copy(dst_hbm.at[pl.ds(0, SCH)], ix, sem).wait()
            pltpu.make_async_copy(msg_hbm.at[:, pl.ds(0, SCH)], col,
                                  sem).wait()

        def body(i, carry):
            cb = eg + (2 * i + 1) * NEG
            cn = jnp.minimum(eg + (2 * i + 2) * NEG, NCHT - 1)
            wait_fetch(ix_a, col_a, sem_a)
            fetch(cb, ix_b, col_b, sem_b)
            scatter(ix_a, col_a)
            wait_fetch(ix_b, col_b, sem_b)
            fetch(cn, ix_a, col_a, sem_a)
            scatter(ix_b, col_b)
            return carry

        lax.fori_loop(0, npairs, body, 0)
        wait_fetch(ix_a, col_a, sem_a)

        @pl.when(eg < ntail)
        def _tail():
            scatter(ix_a, col_a)

        pltpu.sync_copy(acc, out_hbm.at[wid])

    return k(msg_t, dst, zeros_acc)


def _tc_edge(gd, gs, rr, eap, wd, ws, wea, wdr, b1, w2, b2, wc1, bc1,
             wc2, bc2):
    nb = E // BE

    def body(gd_ref, gs_ref, rr_ref, ea_ref, wd_ref, ws_ref, wea_ref,
             wdr_ref, b1_ref, w2_ref, b2_ref, wc1_ref, bc1_ref, wc2_ref,
             bc2_ref, out_ref):
        rrt = rr_ref[...].T
        rel = rrt[:, :POS]
        rd = rrt[:, POS:POS + 1]
        h = (jnp.dot(gd_ref[...], wd_ref[...],
                     preferred_element_type=jnp.float32)
             + jnp.dot(gs_ref[...], ws_ref[...],
                       preferred_element_type=jnp.float32)
             + jnp.dot(ea_ref[...], wea_ref[...],
                       preferred_element_type=jnp.float32)
             + rd * wdr_ref[...]
             + b1_ref[...])
        h = _silu(h)
        m = _silu(jnp.dot(h, w2_ref[...], preferred_element_type=jnp.float32)
                  + b2_ref[...])
        cw = _silu(jnp.dot(m, wc1_ref[...], preferred_element_type=jnp.float32)
                   + bc1_ref[...])
        cw = jnp.dot(cw, wc2_ref[...], preferred_element_type=jnp.float32) \
            + bc2_ref[...]
        out_ref[...] = jnp.concatenate(
            [m, cw * rel, jnp.zeros((BE, MSGW - 19), jnp.float32)],
            axis=1).T

    full = lambda shape: pl.BlockSpec(shape, lambda i: (0,) * len(shape))
    return pl.pallas_call(
        body,
        grid=(nb,),
        in_specs=[
            pl.BlockSpec((BE, F), lambda i: (i, 0)),
            pl.BlockSpec((BE, F), lambda i: (i, 0)),
            pl.BlockSpec((4, BE), lambda i: (0, i)),
            pl.BlockSpec((BE, 8), lambda i: (i, 0)),
            full((F, H1)), full((F, H1)), full((8, H1)), full((1, H1)),
            full((1, H1)), full((H1, 16)), full((1, 16)),
            full((16, 64)), full((1, 64)), full((64, 1)), full((1, 1)),
        ],
        out_specs=pl.BlockSpec((MSGW, BE), lambda i: (0, i)),
        out_shape=jax.ShapeDtypeStruct((MSGW, E), jnp.float32),
    )(gd, gs, rr, eap, wd, ws, wea, wdr, b1, w2, b2, wc1, bc1, wc2, bc2)


def _tc_node(feats, coors2d, acc3, wn1, bn1, wn2, bn2):
    nb = -(-N // BN)

    def body(f_ref, c_ref, acc_ref, wn1_ref, bn1_ref, wn2_ref, bn2_ref,
             fo_ref, co_ref):
        a0 = acc_ref[0]
        a1 = acc_ref[1]
        for g in range(1, NEG):
            a0 = a0 + acc_ref[g * NCG]
            a1 = a1 + acc_ref[g * NCG + 1]
        a = jnp.concatenate([a0.T, a1.T], axis=1)
        feats = f_ref[...]
        nin = jnp.concatenate([feats, a[:, :16]], axis=1)
        hid = _silu(jnp.dot(nin, wn1_ref[...],
                            preferred_element_type=jnp.float32) + bn1_ref[...])
        hid = jnp.dot(hid, wn2_ref[...],
                      preferred_element_type=jnp.float32) + bn2_ref[...]
        fo_ref[...] = feats + hid
        co_ref[...] = c_ref[...] + jnp.concatenate(
            [a[:, 16:19], jnp.zeros((BN, 1), jnp.float32)], axis=1).T

    full = lambda shape: pl.BlockSpec(shape, lambda i: (0,) * len(shape))
    return pl.pallas_call(
        body,
        grid=(nb,),
        in_specs=[
            pl.BlockSpec((BN, F), lambda i: (i, 0)),
            pl.BlockSpec((4, BN), lambda i: (0, i)),
            pl.BlockSpec((NW, CCG, BN), lambda i: (0, 0, i)),
            full((F + 16, 2 * F)), full((1, 2 * F)),
            full((2 * F, F)), full((1, F)),
        ],
        out_specs=[pl.BlockSpec((BN, F), lambda i: (i, 0)),
                   pl.BlockSpec((4, BN), lambda i: (0, i))],
        out_shape=[jax.ShapeDtypeStruct((N, F), jnp.float32),
                   jax.ShapeDtypeStruct((4, NPAD), jnp.float32)],
    )(feats, coors2d, acc3, wn1, bn1, wn2, bn2)


def _pad_weights(p):
    w1 = jnp.pad(p["We1"], ((0, 0), (0, H1 - p["We1"].shape[1])))
    wd = w1[:F]
    ws = w1[F:2 * F]
    wea = jnp.pad(w1[2 * F:2 * F + 4], ((0, 4), (0, 0)))
    wdr = w1[2 * F + 4:2 * F + 5]
    b1 = jnp.pad(p["be1"], (0, H1 - p["be1"].shape[0])).reshape(1, H1)
    w2 = jnp.pad(p["We2"], ((0, H1 - p["We2"].shape[0]), (0, 0)))
    return dict(wd=wd, ws=ws, wea=wea, wdr=wdr, b1=b1, w2=w2,
                b2=p["be2"].reshape(1, -1),
                wc1=p["Wc1"], bc1=p["bc1"].reshape(1, -1),
                wc2=p["Wc2"], bc2=p["bc2"].reshape(1, -1),
                wn1=p["Wn1"], bn1=p["bn1"].reshape(1, -1),
                wn2=p["Wn2"], bn2=p["bn2"].reshape(1, -1))


def kernel(x, edge_index, batch, edge_attr, params):
    src = edge_index[0]
    dst = edge_index[1]
    feats = x[:, POS:]
    coors2d = jnp.pad(
        jnp.concatenate([x[:, :POS].T, jnp.zeros((1, N), jnp.float32)],
                        axis=0), ((0, 0), (0, NPAD - N)))
    eap = jnp.pad(edge_attr, ((0, 0), (0, 4)))
    zeros_acc = jnp.zeros((CCG * NPAD,), jnp.float32)
    for p in params:
        w = _pad_weights(p)
        gd, gs, rr = _sc_gather(feats, coors2d.reshape(4 * NPAD), dst, src)
        msg_t = _tc_edge(gd, gs, rr, eap, w["wd"], w["ws"], w["wea"],
                         w["wdr"], w["b1"], w["w2"], w["b2"], w["wc1"],
                         w["bc1"], w["wc2"], w["bc2"])
        acc = _sc_scatter(msg_t, dst, zeros_acc)
        acc3 = acc.reshape(NW, CCG, NPAD)
        feats, coors2d = _tc_node(feats, coors2d, acc3, w["wn1"], w["bn1"],
                                  w["wn2"], w["bn2"])
    return jnp.concatenate([coors2d[:POS, :N].T, feats], axis=1)


# speedup vs baseline: 1.0115x; 1.0011x over previous
import functools

import jax
import jax.numpy as jnp
from jax import lax
from jax.experimental import pallas as pl
from jax.experimental.pallas import tpu as pltpu
from jax.experimental.pallas import tpu_sc as plsc

N = 10000
E = 320000
F = 128
POS = 3
MSGW = 32
H1 = 528
SCH = 128
NCHT = E // SCH
NW = 32
NEG = 16
NCG = 2
CCG = 10
NPAD = 10240
BE = 3200
BN = 2048


def _silu(v):
    return v * jax.nn.sigmoid(v)


def _sc_gather(feats, coors_flat, dst, src):
    info = plsc.get_sparse_core_info()
    nc = info.num_cores
    mesh = plsc.VectorSubcoreMesh(core_axis_name="c", subcore_axis_name="s")
    npairs = (NCHT // NW) // 2
    ntail = NCHT - NW * 2 * npairs
    ngr = SCH // 16

    @functools.partial(
        pl.kernel,
        mesh=mesh,
        compiler_params=pltpu.CompilerParams(needs_layout_passes=False),
        out_type=[jax.ShapeDtypeStruct((E, F), jnp.float32),
                  jax.ShapeDtypeStruct((E, F), jnp.float32),
                  jax.ShapeDtypeStruct((4, E), jnp.float32)],
        scratch_types=[pltpu.VMEM((4 * NPAD,), jnp.float32)]
        + [pltpu.VMEM((SCH,), jnp.int32) for _ in range(4)]
        + [pltpu.VMEM((SCH, F), jnp.float32) for _ in range(4)]
        + [pltpu.VMEM((4, SCH), jnp.float32) for _ in range(2)]
        + [pltpu.SemaphoreType.DMA for _ in range(4)],
    )
    def k(feats_hbm, coor_hbm, dst_hbm, src_hbm, gd_hbm, gs_hbm, rr_hbm,
          coor_v, ixd_a, ixs_a, ixd_b, ixs_b, rod_a, ros_a, rod_b, ros_b,
          rr_a, rr_b, sem_a, sem_b, sem_wa, sem_wb):
        wid = lax.axis_index("s") * nc + lax.axis_index("c")
        pltpu.sync_copy(coor_hbm, coor_v)

        def relrd(ixd, ixs, rr_v):
            for g in range(ngr):
                dvec = ixd[pl.ds(g * 16, 16)]
                svec = ixs[pl.ds(g * 16, 16)]
                rd = jnp.zeros((16,), jnp.float32)
                for d in range(POS):
                    cd = plsc.load_gather(coor_v, [dvec + d * NPAD])
                    cs = plsc.load_gather(coor_v, [svec + d * NPAD])
                    rel = cs - cd
                    rr_v[d, pl.ds(g * 16, 16)] = rel
                    rd = rd + rel * rel
                rr_v[POS, pl.ds(g * 16, 16)] = rd

        def fetch(chunk, ixd, ixs, rod, ros, sem):
            off = chunk * SCH
            pltpu.sync_copy(dst_hbm.at[pl.ds(off, SCH)], ixd)
            pltpu.sync_copy(src_hbm.at[pl.ds(off, SCH)], ixs)
            cp_d = pltpu.async_copy(feats_hbm.at[ixd], rod, sem)
            cp_s = pltpu.async_copy(feats_hbm.at[ixs], ros, sem)
            return cp_d, cp_s

        def flush(chunk, rod, ros, rr_v, sem_w):
            off = chunk * SCH
            wd = pltpu.async_copy(rod, gd_hbm.at[pl.ds(off, SCH)], sem_w)
            ws = pltpu.async_copy(ros, gs_hbm.at[pl.ds(off, SCH)], sem_w)
            pltpu.sync_copy(rr_v, rr_hbm.at[:, pl.ds(off, SCH)])
            return wd, ws

        def body(i, carry):
            ca = wid + (2 * i) * NW
            cb = wid + (2 * i + 1) * NW
            ga_d, ga_s = fetch(ca, ixd_a, ixs_a, rod_a, ros_a, sem_a)
            gb_d, gb_s = fetch(cb, ixd_b, ixs_b, rod_b, ros_b, sem_b)
            relrd(ixd_a, ixs_a, rr_a)
            ga_d.wait()
            ga_s.wait()
            wa_d, wa_s = flush(ca, rod_a, ros_a, rr_a, sem_wa)
            relrd(ixd_b, ixs_b, rr_b)
            gb_d.wait()
            gb_s.wait()
            wb_d, wb_s = flush(cb, rod_b, ros_b, rr_b, sem_wb)
            wa_d.wait()
            wa_s.wait()
            wb_d.wait()
            wb_s.wait()
            return carry

        lax.fori_loop(0, npairs, body, 0)

        @pl.when(wid < ntail)
        def _tail():
            ct = NW * 2 * npairs + wid
            ga_d, ga_s = fetch(ct, ixd_a, ixs_a, rod_a, ros_a, sem_a)
            relrd(ixd_a, ixs_a, rr_a)
            ga_d.wait()
            ga_s.wait()
            wa_d, wa_s = flush(ct, rod_a, ros_a, rr_a, sem_wa)
            wa_d.wait()
            wa_s.wait()

    return k(feats, coors_flat, dst, src)


def _sc_scatter(msg_t, dst, zeros_acc):
    info = plsc.get_sparse_core_info()
    nc = info.num_cores
    mesh = plsc.VectorSubcoreMesh(core_axis_name="c", subcore_axis_name="s")
    ngr = SCH // 16
    npairs = (NCHT // NEG) // 2
    ntail = NCHT - NEG * 2 * npairs

    @functools.partial(
        pl.kernel,
        mesh=mesh,
        compiler_params=pltpu.CompilerParams(needs_layout_passes=False),
        out_type=jax.ShapeDtypeStruct((NW, CCG * NPAD), jnp.float32),
        scratch_types=[pltpu.VMEM((SCH,), jnp.int32),
                       pltpu.VMEM((SCH,), jnp.int32),
                       pltpu.VMEM((MSGW, SCH), jnp.float32),
                       pltpu.VMEM((MSGW, SCH), jnp.float32),
                       pltpu.VMEM((CCG * NPAD,), jnp.float32),
                       pltpu.SemaphoreType.DMA,
                       pltpu.SemaphoreType.DMA],
    )
    def k(msg_hbm, dst_hbm, z_hbm, out_hbm, ix_a, ix_b, col_a, col_b, acc,
          sem_a, sem_b):
        wid = lax.axis_index("s") * nc + lax.axis_index("c")
        eg = wid // NCG
        cg = wid % NCG
        cbase = cg * CCG
        pltpu.sync_copy(z_hbm, acc)

        def fetch(chunk, ix, col, sem):
            off = chunk * SCH
            ci = pltpu.async_copy(dst_hbm.at[pl.ds(off, SCH)], ix, sem)
            cm = pltpu.async_copy(msg_hbm.at[:, pl.ds(off, SCH)], col, sem)
            return ci, cm

        def scatter(ix, col):
            for g in range(ngr):
                dvec = ix[pl.ds(g * 16, 16)]
                for c in range(CCG):
                    vals = col[cbase + c, pl.ds(g * 16, 16)]
                    plsc.addupdate_scatter(acc, [dvec + c * NPAD], vals)

        fetch(eg, ix_a, col_a, sem_a)

        def wait_fetch(ix, col, sem):
            pltpu.make_async_copy(dst_hbm.at[pl.ds(0, SCH)], ix, sem).wait()
            pltpu.make_async_copy(msg_hbm.at[:, pl.ds(0, SCH)], col,
                                  sem).wait()

        def body(i, carry):
            cb = eg + (2 * i + 1) * NEG
            cn = jnp.minimum(eg + (2 * i + 2) * NEG, NCHT - 1)
            wait_fetch(ix_a, col_a, sem_a)
            fetch(cb, ix_b, col_b, sem_b)
            scatter(ix_a, col_a)
            wait_fetch(ix_b, col_b, sem_b)
            fetch(cn, ix_a, col_a, sem_a)
            scatter(ix_b, col_b)
            return carry

        lax.fori_loop(0, npairs, body, 0)
        wait_fetch(ix_a, col_a, sem_a)

        @pl.when(eg < ntail)
        def _tail():
            scatter(ix_a, col_a)

        pltpu.sync_copy(acc, out_hbm.at[wid])

    return k(msg_t, dst, zeros_acc)


def _tc_edge(gd, gs, rr, eap, wd, ws, wea, wdr, b1, w2, b2, wc1, bc1,
             wc2, bc2):
    nb = E // BE

    def body(gd_ref, gs_ref, rr_ref, ea_ref, wd_ref, ws_ref, wea_ref,
             wdr_ref, b1_ref, w2_ref, b2_ref, wc1_ref, bc1_ref, wc2_ref,
             bc2_ref, out_ref):
        rrt = rr_ref[...].T
        rel = rrt[:, :POS]
        rd = rrt[:, POS:POS + 1]
        h = (jnp.dot(gd_ref[...], wd_ref[...],
                     preferred_element_type=jnp.float32)
             + jnp.dot(gs_ref[...], ws_ref[...],
                       preferred_element_type=jnp.float32)
             + jnp.dot(ea_ref[...], wea_ref[...],
                       preferred_element_type=jnp.float32)
             + rd * wdr_ref[...]
             + b1_ref[...])
        h = _silu(h)
        m = _silu(jnp.dot(h, w2_ref[...], preferred_element_type=jnp.float32)
                  + b2_ref[...])
        cw = _silu(jnp.dot(m, wc1_ref[...], preferred_element_type=jnp.float32)
                   + bc1_ref[...])
        cw = jnp.dot(cw, wc2_ref[...], preferred_element_type=jnp.float32) \
            + bc2_ref[...]
        out_ref[...] = jnp.concatenate(
            [m, cw * rel, jnp.zeros((BE, MSGW - 19), jnp.float32)],
            axis=1).T

    full = lambda shape: pl.BlockSpec(shape, lambda i: (0,) * len(shape))
    return pl.pallas_call(
        body,
        grid=(nb,),
        in_specs=[
            pl.BlockSpec((BE, F), lambda i: (i, 0)),
            pl.BlockSpec((BE, F), lambda i: (i, 0)),
            pl.BlockSpec((4, BE), lambda i: (0, i)),
            pl.BlockSpec((BE, 8), lambda i: (i, 0)),
            full((F, H1)), full((F, H1)), full((8, H1)), full((1, H1)),
            full((1, H1)), full((H1, 16)), full((1, 16)),
            full((16, 64)), full((1, 64)), full((64, 1)), full((1, 1)),
        ],
        out_specs=pl.BlockSpec((MSGW, BE), lambda i: (0, i)),
        out_shape=jax.ShapeDtypeStruct((MSGW, E), jnp.float32),
    )(gd, gs, rr, eap, wd, ws, wea, wdr, b1, w2, b2, wc1, bc1, wc2, bc2)


def _tc_node(feats, coors2d, acc3, wn1, bn1, wn2, bn2):
    nb = -(-N // BN)

    def body(f_ref, c_ref, acc_ref, wn1_ref, bn1_ref, wn2_ref, bn2_ref,
             fo_ref, co_ref):
        a0 = acc_ref[0]
        a1 = acc_ref[1]
        for g in range(1, NEG):
            a0 = a0 + acc_ref[g * NCG]
            a1 = a1 + acc_ref[g * NCG + 1]
        a = jnp.concatenate([a0.T, a1.T], axis=1)
        feats = f_ref[...]
        nin = jnp.concatenate([feats, a[:, :16]], axis=1)
        hid = _silu(jnp.dot(nin, wn1_ref[...],
                            preferred_element_type=jnp.float32) + bn1_ref[...])
        hid = jnp.dot(hid, wn2_ref[...],
                      preferred_element_type=jnp.float32) + bn2_ref[...]
        fo_ref[...] = feats + hid
        co_ref[...] = c_ref[...] + jnp.concatenate(
            [a[:, 16:19], jnp.zeros((BN, 1), jnp.float32)], axis=1).T

    full = lambda shape: pl.BlockSpec(shape, lambda i: (0,) * len(shape))
    return pl.pallas_call(
        body,
        grid=(nb,),
        in_specs=[
            pl.BlockSpec((BN, F), lambda i: (i, 0)),
            pl.BlockSpec((4, BN), lambda i: (0, i)),
            pl.BlockSpec((NW, CCG, BN), lambda i: (0, 0, i)),
            full((F + 16, 2 * F)), full((1, 2 * F)),
            full((2 * F, F)), full((1, F)),
        ],
        out_specs=[pl.BlockSpec((BN, F), lambda i: (i, 0)),
                   pl.BlockSpec((4, BN), lambda i: (0, i))],
        out_shape=[jax.ShapeDtypeStruct((N, F), jnp.float32),
                   jax.ShapeDtypeStruct((4, NPAD), jnp.float32)],
    )(feats, coors2d, acc3, wn1, bn1, wn2, bn2)


def _pad_weights(p):
    w1 = jnp.pad(p["We1"], ((0, 0), (0, H1 - p["We1"].shape[1])))
    wd = w1[:F]
    ws = w1[F:2 * F]
    wea = jnp.pad(w1[2 * F:2 * F + 4], ((0, 4), (0, 0)))
    wdr = w1[2 * F + 4:2 * F + 5]
    b1 = jnp.pad(p["be1"], (0, H1 - p["be1"].shape[0])).reshape(1, H1)
    w2 = jnp.pad(p["We2"], ((0, H1 - p["We2"].shape[0]), (0, 0)))
    return dict(wd=wd, ws=ws, wea=wea, wdr=wdr, b1=b1, w2=w2,
                b2=p["be2"].reshape(1, -1),
                wc1=p["Wc1"], bc1=p["bc1"].reshape(1, -1),
                wc2=p["Wc2"], bc2=p["bc2"].reshape(1, -1),
                wn1=p["Wn1"], bn1=p["bn1"].reshape(1, -1),
                wn2=p["Wn2"], bn2=p["bn2"].reshape(1, -1))


def kernel(x, edge_index, batch, edge_attr, params):
    src = edge_index[0]
    dst = edge_index[1]
    feats = x[:, POS:]
    coors2d = jnp.pad(
        jnp.concatenate([x[:, :POS].T, jnp.zeros((1, N), jnp.float32)],
                        axis=0), ((0, 0), (0, NPAD - N)))
    eap = jnp.pad(edge_attr, ((0, 0), (0, 4)))
    zeros_acc = jnp.zeros((CCG * NPAD,), jnp.float32)
    for p in params:
        w = _pad_weights(p)
        gd, gs, rr = _sc_gather(feats, coors2d.reshape(4 * NPAD), dst, src)
        msg_t = _tc_edge(gd, gs, rr, eap, w["wd"], w["ws"], w["wea"],
                         w["wdr"], w["b1"], w["w2"], w["b2"], w["wc1"],
                         w["bc1"], w["wc2"], w["bc2"])
        acc = _sc_scatter(msg_t, dst, zeros_acc)
        acc3 = acc.reshape(NW, CCG, NPAD)
        feats, coors2d = _tc_node(feats, coors2d, acc3, w["wn1"], w["bn1"],
                                  w["wn2"], w["bn2"])
    return jnp.concatenate([coors2d[:POS, :N].T, feats], axis=1)
